# Initial kernel scaffold; baseline (speedup 1.0000x reference)
import jax
import jax.numpy as jnp
from jax.experimental import pallas as pl


def kernel(abundances, temperature, cr_rate, fuv_rate, incidence, alpha, beta, gamma, cr_coef, fuv_coef, reac_idx, species_idx):
    raise NotImplementedError("write your pallas kernel here")


# speedup vs baseline: 6.8225x; 6.8225x over previous
import functools

import jax
import jax.numpy as jnp
from jax.experimental import pallas as pl
from jax.experimental.pallas import tpu as pltpu

N_SPECIES = 1024
N_REACTIONS = 65536
R_BLOCK = 512


def _fused_kernel(t_ref, cr_ref, fuv_ref, ab_ref, al_ref, be_ref, ga_ref,
                  cc_ref, fc_ref, ra_ref, rb_ref, sa_ref, sb_ref, inc_ref,
                  out_ref, *, r_block, n_pairs):
    k = pl.program_id(0)
    t = t_ref[0, 0]
    cr = cr_ref[0, 0]
    fuv = fuv_ref[0, 0]

    rates0 = (al_ref[0:1, :] * jnp.exp(be_ref[0:1, :] * jnp.log(t / 300.0)
                                       - ga_ref[0:1, :] / t)
              + cc_ref[0:1, :] * cr + fc_ref[0:1, :] * fuv)

    rw = jnp.concatenate([ra_ref[0:1, :], rb_ref[0:1, :]], axis=1)
    sw = jnp.concatenate([sa_ref[0:1, :], sb_ref[0:1, :]], axis=1)
    w = 4 * r_block

    la = jnp.log(ab_ref[0:1, :])
    sp_iota = jax.lax.broadcasted_iota(jnp.int32, (N_SPECIES, w), 0)
    oh_g = jnp.where(sp_iota == sw, 1.0, 0.0)
    v = jax.lax.dot_general(la, oh_g, (((1,), (0,)), ((), ())),
                            preferred_element_type=jnp.float32)

    lo = k * r_block
    pos = jax.lax.broadcasted_iota(jnp.int32, (1, w), 1)
    mask = (rw >= lo) & (rw < lo + r_block) & ((k > 0) | (pos >= 2 * r_block))
    v = jnp.where(mask, v, 0.0)

    r_iota = jax.lax.broadcasted_iota(jnp.int32, (r_block, w), 0) + lo
    oh_s = jnp.where(r_iota == rw, 1.0, 0.0)
    g = jax.lax.dot_general(v, oh_s, (((1,), (1,)), ((), ())),
                            preferred_element_type=jnp.float32)

    rates = rates0 * jnp.exp(g)

    contrib = jax.lax.dot_general(inc_ref[:, :], rates, (((1,), (1,)), ((), ())),
                                  preferred_element_type=jnp.float32)

    @pl.when(k == 0)
    def _init():
        out_ref[:, :] = contrib

    @pl.when(k > 0)
    def _acc():
        out_ref[:, :] += contrib


def kernel(abundances, temperature, cr_rate, fuv_rate, incidence, alpha, beta,
           gamma, cr_coef, fuv_coef, reac_idx, species_idx):
    r = R_BLOCK
    nb = N_REACTIONS // r
    pb = 2 * r
    n_pairs = reac_idx.shape[0]
    deficit = 2 * N_REACTIONS - n_pairs
    if deficit > pb:
        raise ValueError("pair-list deficit exceeds one pair block; "
                         "increase R_BLOCK")

    l_pad = nb * pb
    pad = l_pad - n_pairs
    rw = jnp.pad(reac_idx.astype(jnp.int32), (0, pad),
                 constant_values=N_REACTIONS).reshape(1, l_pad)
    sw = jnp.pad(species_idx.astype(jnp.int32), (0, pad),
                 constant_values=0).reshape(1, l_pad)

    row = lambda x: x.reshape(1, -1)
    scl = lambda x: x.reshape(1, 1).astype(jnp.float32)

    pair_a = pl.BlockSpec((1, pb), lambda k: (0, jnp.maximum(k - 1, 0)))
    pair_b = pl.BlockSpec((1, pb), lambda k: (0, k))
    param = pl.BlockSpec((1, r), lambda k: (0, k))
    whole = lambda shape: pl.BlockSpec(shape, lambda k: (0, 0))

    out = pl.pallas_call(
        functools.partial(_fused_kernel, r_block=r, n_pairs=n_pairs),
        grid=(nb,),
        in_specs=[
            whole((1, 1)), whole((1, 1)), whole((1, 1)),
            whole((1, N_SPECIES)),
            param, param, param, param, param,
            pair_a, pair_b, pair_a, pair_b,
            pl.BlockSpec((N_SPECIES, r), lambda k: (0, k)),
        ],
        out_specs=pl.BlockSpec((N_SPECIES, 1), lambda k: (0, 0)),
        out_shape=jax.ShapeDtypeStruct((N_SPECIES, 1), jnp.float32),
        compiler_params=pltpu.CompilerParams(
            dimension_semantics=("arbitrary",),
        ),
    )(scl(temperature), scl(cr_rate), scl(fuv_rate), row(abundances),
      row(alpha), row(beta), row(gamma), row(cr_coef), row(fuv_coef),
      rw, rw, sw, sw, incidence)
    return out.reshape(N_SPECIES)


# speedup vs baseline: 8.8196x; 1.2927x over previous
import functools

import jax
import jax.numpy as jnp
from jax.experimental import pallas as pl
from jax.experimental.pallas import tpu as pltpu

N_SPECIES = 1024
N_REACTIONS = 65536
R_BLOCK = 512


def _fused_kernel(t_ref, cr_ref, fuv_ref, ab_ref, al_ref, be_ref, ga_ref,
                  cc_ref, fc_ref, ra_ref, rb_ref, rc_ref, sa_ref, sb_ref,
                  sc_ref, inc_ref, out_ref, *, r_block):
    k = pl.program_id(0)
    t = t_ref[0, 0]
    cr = cr_ref[0, 0]
    fuv = fuv_ref[0, 0]
    pb2 = r_block
    w = 3 * pb2

    rates0 = (al_ref[0:1, :] * jnp.exp(be_ref[0:1, :] * jnp.log(t / 300.0)
                                       - ga_ref[0:1, :] / t)
              + cc_ref[0:1, :] * cr + fc_ref[0:1, :] * fuv)

    rw = jnp.concatenate([ra_ref[0:1, :], rb_ref[0:1, :], rc_ref[0:1, :]],
                         axis=1)
    sw = jnp.concatenate([sa_ref[0:1, :], sb_ref[0:1, :], sc_ref[0:1, :]],
                         axis=1)

    la = jnp.log(ab_ref[:, :])
    iota32 = jax.lax.broadcasted_iota(jnp.int32, (32, w), 0)
    oh_lo = jnp.where(iota32 == (sw & 31), 1.0, 0.0)
    cols = jax.lax.dot_general(la, oh_lo, (((1,), (0,)), ((), ())),
                               preferred_element_type=jnp.float32)
    f = jnp.sum(jnp.where(iota32 == (sw >> 5), cols, 0.0),
                axis=0, keepdims=True)

    pos = jax.lax.broadcasted_iota(jnp.int32, (1, w), 1)
    v = jnp.where((k > 0) | (pos >= pb2), f, 0.0)

    off = rw - k * r_block
    iota16 = jax.lax.broadcasted_iota(jnp.int32, (16, w), 0)
    bv = jnp.where(iota16 == (off >> 5), v, 0.0)
    oh_lo2 = jnp.where(iota32 == (off & 31), 1.0, 0.0)
    g = jax.lax.dot_general(bv, oh_lo2, (((1,), (1,)), ((), ())),
                            preferred_element_type=jnp.float32)

    e = jnp.exp(g)
    tiled = jnp.tile(e, (1, 16))
    lane = jax.lax.broadcasted_iota(jnp.int32, (16, r_block), 1)
    rows = jax.lax.broadcasted_iota(jnp.int32, (16, r_block), 0)
    flat = jnp.sum(jnp.where(rows == (lane >> 5), tiled, 0.0),
                   axis=0, keepdims=True)

    rates = rates0 * flat

    contrib = jax.lax.dot_general(inc_ref[:, :], rates,
                                  (((1,), (1,)), ((), ())),
                                  preferred_element_type=jnp.float32)

    @pl.when(k == 0)
    def _init():
        out_ref[:, :] = contrib

    @pl.when(k > 0)
    def _acc():
        out_ref[:, :] += contrib


def kernel(abundances, temperature, cr_rate, fuv_rate, incidence, alpha, beta,
           gamma, cr_coef, fuv_coef, reac_idx, species_idx):
    r = R_BLOCK
    assert r == 512, "factorized scatter radix assumes R_BLOCK == 512"
    nb = N_REACTIONS // r
    pb2 = r
    n_pairs = reac_idx.shape[0]
    deficit = 2 * N_REACTIONS - n_pairs
    if deficit > pb2:
        raise ValueError("pair-list deficit exceeds a half-width pair block")

    l_pad = 2 * nb * pb2
    pad = l_pad - n_pairs
    rw = jnp.pad(reac_idx.astype(jnp.int32), (0, pad),
                 constant_values=N_REACTIONS).reshape(1, l_pad)
    sw = jnp.pad(species_idx.astype(jnp.int32), (0, pad),
                 constant_values=0).reshape(1, l_pad)

    row = lambda x: x.reshape(1, -1)
    scl = lambda x: x.reshape(1, 1).astype(jnp.float32)

    pair_a = pl.BlockSpec((1, pb2), lambda k: (0, jnp.maximum(2 * k - 1, 0)))
    pair_b = pl.BlockSpec((1, pb2), lambda k: (0, 2 * k))
    pair_c = pl.BlockSpec((1, pb2), lambda k: (0, 2 * k + 1))
    param = pl.BlockSpec((1, r), lambda k: (0, k))
    whole = lambda shape: pl.BlockSpec(shape, lambda k: (0, 0))

    out = pl.pallas_call(
        functools.partial(_fused_kernel, r_block=r),
        grid=(nb,),
        in_specs=[
            whole((1, 1)), whole((1, 1)), whole((1, 1)),
            whole((32, 32)),
            param, param, param, param, param,
            pair_a, pair_b, pair_c, pair_a, pair_b, pair_c,
            pl.BlockSpec((N_SPECIES, r), lambda k: (0, k)),
        ],
        out_specs=pl.BlockSpec((N_SPECIES, 1), lambda k: (0, 0)),
        out_shape=jax.ShapeDtypeStruct((N_SPECIES, 1), jnp.float32),
        compiler_params=pltpu.CompilerParams(
            dimension_semantics=("arbitrary",),
        ),
    )(scl(temperature), scl(cr_rate), scl(fuv_rate),
      abundances.reshape(32, 32),
      row(alpha), row(beta), row(gamma), row(cr_coef), row(fuv_coef),
      rw, rw, rw, sw, sw, sw, incidence)
    return out.reshape(N_SPECIES)


# speedup vs baseline: 12.1445x; 1.3770x over previous
import functools

import jax
import jax.numpy as jnp
from jax.experimental import pallas as pl
from jax.experimental.pallas import tpu as pltpu

N_SPECIES = 1024
N_REACTIONS = 65536
R_BLOCK = 1024


def _fused_kernel(t_ref, cr_ref, fuv_ref, ab_ref, al_ref, be_ref, ga_ref,
                  cc_ref, fc_ref, ra_ref, rb_ref, rc_ref, sa_ref, sb_ref,
                  sc_ref, inc_ref, out_ref, *, r_block):
    k = pl.program_id(0)
    t = t_ref[0, 0]
    cr = cr_ref[0, 0]
    fuv = fuv_ref[0, 0]
    pb2 = r_block
    w = 3 * pb2

    rates0 = (al_ref[0:1, :] * jnp.exp(be_ref[0:1, :] * jnp.log(t / 300.0)
                                       - ga_ref[0:1, :] / t)
              + cc_ref[0:1, :] * cr + fc_ref[0:1, :] * fuv)

    rw = jnp.concatenate([ra_ref[0:1, :], rb_ref[0:1, :], rc_ref[0:1, :]],
                         axis=1)
    sw = jnp.concatenate([sa_ref[0:1, :], sb_ref[0:1, :], sc_ref[0:1, :]],
                         axis=1)

    la = jnp.log(ab_ref[:, :])
    iota32 = jax.lax.broadcasted_iota(jnp.int32, (32, w), 0)
    oh_lo = jnp.where(iota32 == (sw & 31), 1.0, 0.0)
    cols = jax.lax.dot_general(la, oh_lo, (((1,), (0,)), ((), ())),
                               preferred_element_type=jnp.float32)
    f = jnp.sum(jnp.where(iota32 == (sw >> 5), cols, 0.0),
                axis=0, keepdims=True)

    pos = jax.lax.broadcasted_iota(jnp.int32, (1, w), 1)
    v = jnp.where((k > 0) | (pos >= pb2), f, 0.0)

    off = rw - k * r_block
    hi_rows = r_block >> 5
    iota_hi = jax.lax.broadcasted_iota(jnp.int32, (hi_rows, w), 0)
    bv = jnp.where(iota_hi == (off >> 5), v, 0.0)
    oh_lo2 = jnp.where(iota32 == (off & 31), 1.0, 0.0)
    g = jax.lax.dot_general(bv, oh_lo2, (((1,), (1,)), ((), ())),
                            preferred_element_type=jnp.float32)

    e = jnp.exp(g)
    tiled = jnp.tile(e, (1, hi_rows))
    lane = jax.lax.broadcasted_iota(jnp.int32, (hi_rows, r_block), 1)
    rows = jax.lax.broadcasted_iota(jnp.int32, (hi_rows, r_block), 0)
    flat = jnp.sum(jnp.where(rows == (lane >> 5), tiled, 0.0),
                   axis=0, keepdims=True)

    rates = rates0 * flat

    contrib = jax.lax.dot_general(inc_ref[:, :], rates,
                                  (((1,), (1,)), ((), ())),
                                  preferred_element_type=jnp.float32)

    @pl.when(k == 0)
    def _init():
        out_ref[:, :] = contrib

    @pl.when(k > 0)
    def _acc():
        out_ref[:, :] += contrib


def kernel(abundances, temperature, cr_rate, fuv_rate, incidence, alpha, beta,
           gamma, cr_coef, fuv_coef, reac_idx, species_idx):
    r = R_BLOCK
    nb = N_REACTIONS // r
    pb2 = r
    n_pairs = reac_idx.shape[0]
    deficit = 2 * N_REACTIONS - n_pairs
    if deficit > pb2:
        raise ValueError("pair-list deficit exceeds a half-width pair block")

    l_pad = 2 * nb * pb2
    pad = l_pad - n_pairs
    rw = jnp.pad(reac_idx.astype(jnp.int32), (0, pad),
                 constant_values=N_REACTIONS).reshape(1, l_pad)
    sw = jnp.pad(species_idx.astype(jnp.int32), (0, pad),
                 constant_values=0).reshape(1, l_pad)

    row = lambda x: x.reshape(1, -1)
    scl = lambda x: x.reshape(1, 1).astype(jnp.float32)

    pair_a = pl.BlockSpec((1, pb2), lambda k: (0, jnp.maximum(2 * k - 1, 0)))
    pair_b = pl.BlockSpec((1, pb2), lambda k: (0, 2 * k))
    pair_c = pl.BlockSpec((1, pb2), lambda k: (0, 2 * k + 1))
    param = pl.BlockSpec((1, r), lambda k: (0, k))
    whole = lambda shape: pl.BlockSpec(shape, lambda k: (0, 0))

    out = pl.pallas_call(
        functools.partial(_fused_kernel, r_block=r),
        grid=(nb,),
        in_specs=[
            whole((1, 1)), whole((1, 1)), whole((1, 1)),
            whole((32, 32)),
            param, param, param, param, param,
            pair_a, pair_b, pair_c, pair_a, pair_b, pair_c,
            pl.BlockSpec((N_SPECIES, r), lambda k: (0, k)),
        ],
        out_specs=pl.BlockSpec((N_SPECIES, 1), lambda k: (0, 0)),
        out_shape=jax.ShapeDtypeStruct((N_SPECIES, 1), jnp.float32),
        compiler_params=pltpu.CompilerParams(
            dimension_semantics=("arbitrary",),
        ),
    )(scl(temperature), scl(cr_rate), scl(fuv_rate),
      abundances.reshape(32, 32),
      row(alpha), row(beta), row(gamma), row(cr_coef), row(fuv_coef),
      rw, rw, rw, sw, sw, sw, incidence)
    return out.reshape(N_SPECIES)


# speedup vs baseline: 14.5353x; 1.1969x over previous
import functools

import jax
import jax.numpy as jnp
from jax.experimental import pallas as pl
from jax.experimental.pallas import tpu as pltpu

N_SPECIES = 1024
N_REACTIONS = 65536
R_BLOCK = 2048


def _fused_kernel(t_ref, cr_ref, fuv_ref, ab_ref, al_ref, be_ref, ga_ref,
                  cc_ref, fc_ref, ra_ref, rb_ref, rc_ref, sa_ref, sb_ref,
                  sc_ref, inc_ref, out_ref, *, r_block):
    k = pl.program_id(0)
    t = t_ref[0, 0]
    cr = cr_ref[0, 0]
    fuv = fuv_ref[0, 0]
    pb2 = r_block
    w = 3 * pb2

    rates0 = (al_ref[0:1, :] * jnp.exp(be_ref[0:1, :] * jnp.log(t / 300.0)
                                       - ga_ref[0:1, :] / t)
              + cc_ref[0:1, :] * cr + fc_ref[0:1, :] * fuv)

    rw = jnp.concatenate([ra_ref[0:1, :], rb_ref[0:1, :], rc_ref[0:1, :]],
                         axis=1)
    sw = jnp.concatenate([sa_ref[0:1, :], sb_ref[0:1, :], sc_ref[0:1, :]],
                         axis=1)

    la = jnp.log(ab_ref[:, :])
    iota32 = jax.lax.broadcasted_iota(jnp.int32, (32, w), 0)
    oh_lo = jnp.where(iota32 == (sw & 31), 1.0, 0.0)
    cols = jax.lax.dot_general(la, oh_lo, (((1,), (0,)), ((), ())),
                               preferred_element_type=jnp.float32)
    f = jnp.sum(jnp.where(iota32 == (sw >> 5), cols, 0.0),
                axis=0, keepdims=True)

    pos = jax.lax.broadcasted_iota(jnp.int32, (1, w), 1)
    v = jnp.where((k > 0) | (pos >= pb2), f, 0.0)

    off = rw - k * r_block
    hi_rows = r_block >> 5
    iota_hi = jax.lax.broadcasted_iota(jnp.int32, (hi_rows, w), 0)
    bv = jnp.where(iota_hi == (off >> 5), v, 0.0)
    oh_lo2 = jnp.where(iota32 == (off & 31), 1.0, 0.0)
    g = jax.lax.dot_general(bv, oh_lo2, (((1,), (1,)), ((), ())),
                            preferred_element_type=jnp.float32)

    e = jnp.exp(g)
    tiled = jnp.tile(e, (1, hi_rows))
    lane = jax.lax.broadcasted_iota(jnp.int32, (hi_rows, r_block), 1)
    rows = jax.lax.broadcasted_iota(jnp.int32, (hi_rows, r_block), 0)
    flat = jnp.sum(jnp.where(rows == (lane >> 5), tiled, 0.0),
                   axis=0, keepdims=True)

    rates = rates0 * flat

    contrib = jax.lax.dot_general(inc_ref[:, :], rates,
                                  (((1,), (1,)), ((), ())),
                                  preferred_element_type=jnp.float32)

    @pl.when(k == 0)
    def _init():
        out_ref[:, :] = contrib

    @pl.when(k > 0)
    def _acc():
        out_ref[:, :] += contrib


def kernel(abundances, temperature, cr_rate, fuv_rate, incidence, alpha, beta,
           gamma, cr_coef, fuv_coef, reac_idx, species_idx):
    r = R_BLOCK
    nb = N_REACTIONS // r
    pb2 = r
    n_pairs = reac_idx.shape[0]
    deficit = 2 * N_REACTIONS - n_pairs
    if deficit > pb2:
        raise ValueError("pair-list deficit exceeds a half-width pair block")

    l_pad = 2 * nb * pb2
    pad = l_pad - n_pairs
    rw = jnp.pad(reac_idx.astype(jnp.int32), (0, pad),
                 constant_values=N_REACTIONS).reshape(1, l_pad)
    sw = jnp.pad(species_idx.astype(jnp.int32), (0, pad),
                 constant_values=0).reshape(1, l_pad)

    row = lambda x: x.reshape(1, -1)
    scl = lambda x: x.reshape(1, 1).astype(jnp.float32)

    pair_a = pl.BlockSpec((1, pb2), lambda k: (0, jnp.maximum(2 * k - 1, 0)))
    pair_b = pl.BlockSpec((1, pb2), lambda k: (0, 2 * k))
    pair_c = pl.BlockSpec((1, pb2), lambda k: (0, 2 * k + 1))
    param = pl.BlockSpec((1, r), lambda k: (0, k))
    whole = lambda shape: pl.BlockSpec(shape, lambda k: (0, 0))

    out = pl.pallas_call(
        functools.partial(_fused_kernel, r_block=r),
        grid=(nb,),
        in_specs=[
            whole((1, 1)), whole((1, 1)), whole((1, 1)),
            whole((32, 32)),
            param, param, param, param, param,
            pair_a, pair_b, pair_c, pair_a, pair_b, pair_c,
            pl.BlockSpec((N_SPECIES, r), lambda k: (0, k)),
        ],
        out_specs=pl.BlockSpec((N_SPECIES, 1), lambda k: (0, 0)),
        out_shape=jax.ShapeDtypeStruct((N_SPECIES, 1), jnp.float32),
        compiler_params=pltpu.CompilerParams(
            dimension_semantics=("arbitrary",),
        ),
    )(scl(temperature), scl(cr_rate), scl(fuv_rate),
      abundances.reshape(32, 32),
      row(alpha), row(beta), row(gamma), row(cr_coef), row(fuv_coef),
      rw, rw, rw, sw, sw, sw, incidence)
    return out.reshape(N_SPECIES)


# speedup vs baseline: 15.2946x; 1.0522x over previous
import functools

import jax
import jax.numpy as jnp
from jax.experimental import pallas as pl
from jax.experimental.pallas import tpu as pltpu

N_SPECIES = 1024
N_REACTIONS = 65536
R_BLOCK = 4096


def _fused_kernel(t_ref, cr_ref, fuv_ref, ab_ref, al_ref, be_ref, ga_ref,
                  cc_ref, fc_ref, ra_ref, rb_ref, rc_ref, sa_ref, sb_ref,
                  sc_ref, inc_ref, out_ref, *, r_block):
    k = pl.program_id(0)
    t = t_ref[0, 0]
    cr = cr_ref[0, 0]
    fuv = fuv_ref[0, 0]
    pb2 = r_block
    w = 3 * pb2

    rates0 = (al_ref[0:1, :] * jnp.exp(be_ref[0:1, :] * jnp.log(t / 300.0)
                                       - ga_ref[0:1, :] / t)
              + cc_ref[0:1, :] * cr + fc_ref[0:1, :] * fuv)

    rw = jnp.concatenate([ra_ref[0:1, :], rb_ref[0:1, :], rc_ref[0:1, :]],
                         axis=1)
    sw = jnp.concatenate([sa_ref[0:1, :], sb_ref[0:1, :], sc_ref[0:1, :]],
                         axis=1)

    la = jnp.log(ab_ref[:, :])
    iota32 = jax.lax.broadcasted_iota(jnp.int32, (32, w), 0)
    oh_lo = jnp.where(iota32 == (sw & 31), 1.0, 0.0)
    cols = jax.lax.dot_general(la, oh_lo, (((1,), (0,)), ((), ())),
                               preferred_element_type=jnp.float32)
    f = jnp.sum(jnp.where(iota32 == (sw >> 5), cols, 0.0),
                axis=0, keepdims=True)

    pos = jax.lax.broadcasted_iota(jnp.int32, (1, w), 1)
    v = jnp.where((k > 0) | (pos >= pb2), f, 0.0)

    off = rw - k * r_block
    hi_rows = r_block >> 5
    iota_hi = jax.lax.broadcasted_iota(jnp.int32, (hi_rows, w), 0)
    bv = jnp.where(iota_hi == (off >> 5), v, 0.0)
    oh_lo2 = jnp.where(iota32 == (off & 31), 1.0, 0.0)
    g = jax.lax.dot_general(bv, oh_lo2, (((1,), (1,)), ((), ())),
                            preferred_element_type=jnp.float32)

    e = jnp.exp(g)
    tiled = jnp.tile(e, (1, hi_rows))
    lane = jax.lax.broadcasted_iota(jnp.int32, (hi_rows, r_block), 1)
    rows = jax.lax.broadcasted_iota(jnp.int32, (hi_rows, r_block), 0)
    flat = jnp.sum(jnp.where(rows == (lane >> 5), tiled, 0.0),
                   axis=0, keepdims=True)

    rates = rates0 * flat

    contrib = jax.lax.dot_general(inc_ref[:, :], rates,
                                  (((1,), (1,)), ((), ())),
                                  preferred_element_type=jnp.float32)

    @pl.when(k == 0)
    def _init():
        out_ref[:, :] = contrib

    @pl.when(k > 0)
    def _acc():
        out_ref[:, :] += contrib


def kernel(abundances, temperature, cr_rate, fuv_rate, incidence, alpha, beta,
           gamma, cr_coef, fuv_coef, reac_idx, species_idx):
    r = R_BLOCK
    nb = N_REACTIONS // r
    pb2 = r
    n_pairs = reac_idx.shape[0]
    deficit = 2 * N_REACTIONS - n_pairs
    if deficit > pb2:
        raise ValueError("pair-list deficit exceeds a half-width pair block")

    l_pad = 2 * nb * pb2
    pad = l_pad - n_pairs
    rw = jnp.pad(reac_idx.astype(jnp.int32), (0, pad),
                 constant_values=N_REACTIONS).reshape(1, l_pad)
    sw = jnp.pad(species_idx.astype(jnp.int32), (0, pad),
                 constant_values=0).reshape(1, l_pad)

    row = lambda x: x.reshape(1, -1)
    scl = lambda x: x.reshape(1, 1).astype(jnp.float32)

    pair_a = pl.BlockSpec((1, pb2), lambda k: (0, jnp.maximum(2 * k - 1, 0)))
    pair_b = pl.BlockSpec((1, pb2), lambda k: (0, 2 * k))
    pair_c = pl.BlockSpec((1, pb2), lambda k: (0, 2 * k + 1))
    param = pl.BlockSpec((1, r), lambda k: (0, k))
    whole = lambda shape: pl.BlockSpec(shape, lambda k: (0, 0))

    out = pl.pallas_call(
        functools.partial(_fused_kernel, r_block=r),
        grid=(nb,),
        in_specs=[
            whole((1, 1)), whole((1, 1)), whole((1, 1)),
            whole((32, 32)),
            param, param, param, param, param,
            pair_a, pair_b, pair_c, pair_a, pair_b, pair_c,
            pl.BlockSpec((N_SPECIES, r), lambda k: (0, k)),
        ],
        out_specs=pl.BlockSpec((N_SPECIES, 1), lambda k: (0, 0)),
        out_shape=jax.ShapeDtypeStruct((N_SPECIES, 1), jnp.float32),
        compiler_params=pltpu.CompilerParams(
            dimension_semantics=("arbitrary",),
        ),
    )(scl(temperature), scl(cr_rate), scl(fuv_rate),
      abundances.reshape(32, 32),
      row(alpha), row(beta), row(gamma), row(cr_coef), row(fuv_coef),
      rw, rw, rw, sw, sw, sw, incidence)
    return out.reshape(N_SPECIES)
